# Initial kernel scaffold; baseline (speedup 1.0000x reference)
import jax
import jax.numpy as jnp
from jax.experimental import pallas as pl


def kernel(tensor):
    raise NotImplementedError("write your pallas kernel here")


# speedup vs baseline: 3.2198x; 3.2198x over previous
import jax
import jax.numpy as jnp
from jax.experimental import pallas as pl
from jax.experimental.pallas import tpu as pltpu

_BS = 256


def _cumsum_body(x_ref, o_ref, carry_ref):
    j = pl.program_id(1)

    @pl.when(j == 0)
    def _():
        carry_ref[...] = jnp.zeros_like(carry_ref)

    x = x_ref[0]
    r = jax.lax.broadcasted_iota(jnp.int32, (_BS, _BS), 0)
    c = jax.lax.broadcasted_iota(jnp.int32, (_BS, _BS), 1)
    tril = (r >= c).astype(jnp.float32)
    cs = jax.lax.dot(tril, x, preferred_element_type=jnp.float32) + carry_ref[...]
    o_ref[0] = cs
    carry_ref[...] = cs[_BS - 1 : _BS, :]


def kernel(tensor):
    B, S, D = tensor.shape
    nb = S // _BS
    return pl.pallas_call(
        _cumsum_body,
        grid=(B, nb),
        in_specs=[pl.BlockSpec((1, _BS, D), lambda b, j: (b, j, 0))],
        out_specs=pl.BlockSpec((1, _BS, D), lambda b, j: (b, j, 0)),
        out_shape=jax.ShapeDtypeStruct(tensor.shape, tensor.dtype),
        scratch_shapes=[pltpu.VMEM((1, D), jnp.float32)],
        compiler_params=pltpu.CompilerParams(
            dimension_semantics=("parallel", "arbitrary")
        ),
    )(tensor)
